# baseline (speedup 1.0000x reference)
import functools

import jax
import jax.numpy as jnp
from jax import lax
from jax.experimental import pallas as pl
from jax.experimental.pallas import tpu as pltpu
from jax.experimental.pallas import tpu_sc as plsc

EPS = 1e-7
SIGMA_BOOST = 2.0
IN_N = 10000
OUT_N = 10000
KP = 25000
N_ADD = 4
SIGMA_SCALE = 0.2
D_IN = 256
DH = 128

PADK = 25600
NBLK1 = PADK // 128
NCAND = 8

NE = 2 * NCAND * PADK
N_TILES = 16
BLK = 64
OUTP = 10240
ROWS_T = OUTP // N_TILES


def _gen_body(p_ref, u_ref, i_ref, j_ref, v_ref):
    p0, p1, p2, p3 = p_ref[0], p_ref[1], p_ref[2], p_ref[3]
    m0 = jax.nn.sigmoid(p0) * (OUT_N - 1.0)
    m1 = jax.nn.sigmoid(p1) * (IN_N - 1.0)
    x = p2 + SIGMA_BOOST
    softplus = jnp.maximum(x, 0.0) + jnp.log1p(jnp.exp(-jnp.abs(x)))
    sig = (softplus + EPS) * (OUT_N * SIGMA_SCALE)
    inv = 1.0 / (EPS + sig)
    f0, c0 = jnp.floor(m0), jnp.ceil(m0)
    f1, c1 = jnp.floor(m1), jnp.ceil(m1)
    cands = [(f0, f1), (f0, c1), (c0, f1), (c0, c1)]
    for smp in range(N_ADD):
        u0 = u_ref[2 * smp] * (1.0 - EPS)
        u1 = u_ref[2 * smp + 1] * (1.0 - EPS)
        cands.append((jnp.floor(u0 * OUT_N), jnp.floor(u1 * IN_N)))
    props = []
    for d0, d1 in cands:
        dx, dy = d0 - m0, d1 - m1
        props.append(jnp.exp(-0.5 * (dx * dx + dy * dy) * inv))
    ssum = sum(props) + NCAND * EPS
    for d, ((d0, d1), pr) in enumerate(zip(cands, props)):
        val = p3 * pr / ssum
        val = jnp.where(d0 == d1, 0.0, val)
        i_ref[d] = d0.astype(jnp.int32)
        j_ref[d] = d1.astype(jnp.int32)
        v_ref[d] = val


def _gen_entries(p_t, u_t):
    return pl.pallas_call(
        _gen_body,
        out_shape=(
            jax.ShapeDtypeStruct((NCAND, NBLK1, 128), jnp.int32),
            jax.ShapeDtypeStruct((NCAND, NBLK1, 128), jnp.int32),
            jax.ShapeDtypeStruct((NCAND, NBLK1, 128), jnp.float32),
        ),
    )(p_t, u_t)


_MESH = plsc.VectorSubcoreMesh(core_axis_name="c", subcore_axis_name="s")
_TILE_ENTRIES = NE // N_TILES
_NBLK2 = _TILE_ENTRIES // BLK
NRING = 4
SBE = 20
NSB = _NBLK2 // SBE


@functools.partial(
    pl.kernel,
    out_type=jax.ShapeDtypeStruct((2, OUTP, DH), jnp.float32),
    mesh=_MESH,
    scratch_types=[
        pltpu.VMEM((2, SBE, BLK), jnp.int32),
        pltpu.VMEM((2, SBE, BLK), jnp.int32),
        pltpu.VMEM((2, SBE * BLK), jnp.float32),
        pltpu.VMEM((NRING, BLK, DH), jnp.float32),
        pltpu.VMEM_SHARED((OUTP, DH), jnp.float32),
        pltpu.SemaphoreType.DMA((NRING,)),
        pltpu.SemaphoreType.DMA((NRING,)),
        pltpu.SemaphoreType.DMA,
    ],
)
def _scatter_kernel(tab_hbm, ii_hbm, jj_hbm, vv_hbm, out_hbm,
                    ii_v, jj_v, vv_v, rows_v, acc_sh, gsem, ssem, isem):
    c = lax.axis_index("c")
    s = lax.axis_index("s")
    coff = c * IN_N

    def _prefetch_superblock(m, slot):
        pltpu.async_copy(ii_hbm.at[s * NSB + m], ii_v.at[slot], isem)
        pltpu.async_copy(jj_hbm.at[s * NSB + m], jj_v.at[slot], isem)
        pltpu.async_copy(vv_hbm.at[s * NSB + m], vv_v.at[slot], isem)

    def _wait_superblock(slot):
        pltpu.make_async_copy(ii_hbm.at[0], ii_v.at[slot], isem).wait()
        pltpu.make_async_copy(jj_hbm.at[0], jj_v.at[slot], isem).wait()
        pltpu.make_async_copy(vv_hbm.at[0], vv_v.at[slot], isem).wait()

    def _adjust_jj(slot):
        def _adj(r, _):
            for g in range(BLK // 16):
                jj_v[slot, r, pl.ds(g * 16, 16)] = (
                    jj_v[slot, r, pl.ds(g * 16, 16)] + coff)
            return 0
        lax.fori_loop(0, SBE, _adj, 0)

    def _drain_scatter(buf):
        pltpu.make_async_copy(
            rows_v.at[buf], acc_sh.at[ii_v.at[0, 0]], ssem.at[buf]).wait()

    def _start_gather(slot, lb, buf):
        pltpu.async_copy(
            tab_hbm.at[jj_v.at[slot, lb]], rows_v.at[buf], gsem.at[buf])

    _prefetch_superblock(0, 0)
    _wait_superblock(0)
    _adjust_jj(0)
    _prefetch_superblock(1, 1)

    zero16 = jnp.zeros((16,), jnp.float32)

    def _zrow(r, _):
        for g in range(DH // 16):
            rows_v[0, r, pl.ds(g * 16, 16)] = zero16
        return 0
    lax.fori_loop(0, BLK, _zrow, 0)
    for t in range(ROWS_T // BLK):
        pltpu.async_copy(rows_v.at[0],
                         acc_sh.at[pl.ds(s * ROWS_T + t * BLK, BLK)],
                         gsem.at[0])
    for t in range(ROWS_T // BLK):
        pltpu.make_async_copy(
            rows_v.at[0], acc_sh.at[pl.ds(s * ROWS_T, BLK)], gsem.at[0]).wait()
    plsc.subcore_barrier()

    _start_gather(0, 0, 0)
    _start_gather(0, 1, 1)

    def _body(b, _):
        lb = lax.rem(b, SBE)
        buf = lax.rem(b, NRING)
        nbuf = lax.rem(b + 1, NRING)
        slot = lax.rem(lax.div(b, SBE), 2)

        @pl.when(jnp.logical_and(lb == 0, b > 0))
        def _boundary():
            for d in (2, 1):
                _drain_scatter(lax.rem(b - d, NRING))
            _wait_superblock(slot)
            _adjust_jj(slot)
            _start_gather(slot, 0, buf)
            _start_gather(slot, 1, nbuf)

            @pl.when(b + SBE < _NBLK2)
            def _():
                _prefetch_superblock(lax.div(b, SBE) + 1, 1 - slot)

        @pl.when(lb >= 2)
        def _():
            _drain_scatter(lax.rem(b + 2, NRING))

        @pl.when(jnp.logical_and(b + 2 < _NBLK2, lax.rem(b + 2, SBE) >= 2))
        def _():
            _start_gather(slot, lax.rem(b + 2, SBE), lax.rem(b + 2, NRING))

        pltpu.make_async_copy(
            tab_hbm.at[jj_v.at[slot, lb]], rows_v.at[buf], gsem.at[buf]).wait()

        for g in range(BLK // 16):
            vvec = vv_v[slot, pl.ds(lb * BLK + g * 16, 16)]
            for l in range(16):
                vb = lax.gather(
                    vvec, jnp.full((16, 1), l, jnp.int32),
                    dimension_numbers=lax.GatherDimensionNumbers(
                        offset_dims=(), collapsed_slice_dims=(0,),
                        start_index_map=(0,)),
                    slice_sizes=(1,),
                    mode=lax.GatherScatterMode.PROMISE_IN_BOUNDS)
                e = g * 16 + l
                for k in range(DH // 16):
                    rows_v[buf, e, pl.ds(k * 16, 16)] = (
                        rows_v[buf, e, pl.ds(k * 16, 16)] * vb)

        pltpu.async_copy(rows_v.at[buf], acc_sh.at[ii_v.at[slot, lb]],
                         ssem.at[buf], add=True)
        return 0
    lax.fori_loop(0, _NBLK2, _body, 0)

    _drain_scatter((_NBLK2 - 2) % NRING)
    _drain_scatter((_NBLK2 - 1) % NRING)
    plsc.subcore_barrier()

    def _row0(t):
        return pl.ds(s * ROWS_T + t * BLK, BLK)

    for t in range(NRING):
        pltpu.async_copy(acc_sh.at[_row0(t)], rows_v.at[t], gsem.at[t])
    for t in range(ROWS_T // BLK):
        sl = t % NRING
        pltpu.make_async_copy(
            acc_sh.at[_row0(t)], rows_v.at[sl], gsem.at[sl]).wait()
        pltpu.async_copy(rows_v.at[sl], out_hbm.at[c, _row0(t)], ssem.at[sl])
        if t + NRING < ROWS_T // BLK:
            pltpu.make_async_copy(
                rows_v.at[sl], out_hbm.at[c, _row0(t)], ssem.at[sl]).wait()
            pltpu.async_copy(acc_sh.at[_row0(t + NRING)], rows_v.at[sl],
                             gsem.at[sl])
    for t in range(ROWS_T // BLK - NRING, ROWS_T // BLK):
        sl = t % NRING
        pltpu.make_async_copy(
            rows_v.at[sl], out_hbm.at[c, _row0(t)], ssem.at[sl]).wait()


def kernel(input, params):
    u = jax.random.uniform(jax.random.key(12345), (KP, N_ADD, 2), dtype=jnp.float32)
    u_t = jnp.pad(u.transpose(1, 2, 0).reshape(2 * N_ADD, KP), ((0, 0), (0, PADK - KP)))
    p_t = jnp.pad(params.T, ((0, 0), (0, PADK - KP)))
    iN, jN, vN = _gen_entries(
        p_t.reshape(4, NBLK1, 128), u_t.reshape(2 * N_ADD, NBLK1, 128))
    iN, jN, vN = iN.reshape(-1), jN.reshape(-1), vN.reshape(-1)
    ii = jnp.concatenate([iN, jN]).reshape(N_TILES * NSB, SBE, BLK)
    jj = jnp.concatenate([jN, iN]).reshape(N_TILES * NSB, SBE, BLK)
    vv = jnp.concatenate([vN, vN]).reshape(N_TILES * NSB, SBE * BLK)
    tab = jnp.concatenate([input[:, :DH], input[:, DH:]], axis=0)
    out2 = _scatter_kernel(tab, ii, jj, vv)
    return jnp.concatenate([out2[0, :OUT_N], out2[1, :OUT_N]], axis=1)


# speedup vs baseline: 1.0033x; 1.0033x over previous
import functools

import jax
import jax.numpy as jnp
from jax import lax
from jax.experimental import pallas as pl
from jax.experimental.pallas import tpu as pltpu
from jax.experimental.pallas import tpu_sc as plsc

EPS = 1e-7
SIGMA_BOOST = 2.0
IN_N = 10000
OUT_N = 10000
KP = 25000
N_ADD = 4
SIGMA_SCALE = 0.2
D_IN = 256
DH = 128

PADK = 25600
NBLK1 = PADK // 128
NCAND = 8

NE = 2 * NCAND * PADK
N_TILES = 16
BLK = 64
OUTP = 10240
ROWS_T = OUTP // N_TILES


def _gen_body(p_ref, u_ref, i_ref, j_ref, v_ref):
    p0, p1, p2, p3 = p_ref[0], p_ref[1], p_ref[2], p_ref[3]
    m0 = jax.nn.sigmoid(p0) * (OUT_N - 1.0)
    m1 = jax.nn.sigmoid(p1) * (IN_N - 1.0)
    x = p2 + SIGMA_BOOST
    softplus = jnp.maximum(x, 0.0) + jnp.log1p(jnp.exp(-jnp.abs(x)))
    sig = (softplus + EPS) * (OUT_N * SIGMA_SCALE)
    inv = 1.0 / (EPS + sig)
    f0, c0 = jnp.floor(m0), jnp.ceil(m0)
    f1, c1 = jnp.floor(m1), jnp.ceil(m1)
    cands = [(f0, f1), (f0, c1), (c0, f1), (c0, c1)]
    for smp in range(N_ADD):
        u0 = u_ref[2 * smp] * (1.0 - EPS)
        u1 = u_ref[2 * smp + 1] * (1.0 - EPS)
        cands.append((jnp.floor(u0 * OUT_N), jnp.floor(u1 * IN_N)))
    props = []
    for d0, d1 in cands:
        dx, dy = d0 - m0, d1 - m1
        props.append(jnp.exp(-0.5 * (dx * dx + dy * dy) * inv))
    ssum = sum(props) + NCAND * EPS
    for d, ((d0, d1), pr) in enumerate(zip(cands, props)):
        val = p3 * pr / ssum
        val = jnp.where(d0 == d1, 0.0, val)
        i_ref[d] = d0.astype(jnp.int32)
        j_ref[d] = d1.astype(jnp.int32)
        v_ref[d] = val


def _gen_entries(p_t, u_t):
    return pl.pallas_call(
        _gen_body,
        out_shape=(
            jax.ShapeDtypeStruct((NCAND, NBLK1, 128), jnp.int32),
            jax.ShapeDtypeStruct((NCAND, NBLK1, 128), jnp.int32),
            jax.ShapeDtypeStruct((NCAND, NBLK1, 128), jnp.float32),
        ),
    )(p_t, u_t)


_MESH = plsc.VectorSubcoreMesh(core_axis_name="c", subcore_axis_name="s")
_TILE_ENTRIES = NE // N_TILES
_NBLK2 = _TILE_ENTRIES // BLK
NRING = 4
SBE = 40
NSB = _NBLK2 // SBE


@functools.partial(
    pl.kernel,
    out_type=jax.ShapeDtypeStruct((2, OUTP, DH), jnp.float32),
    mesh=_MESH,
    scratch_types=[
        pltpu.VMEM((SBE, BLK), jnp.int32),
        pltpu.VMEM((SBE, BLK), jnp.int32),
        pltpu.VMEM((SBE * BLK,), jnp.float32),
        pltpu.VMEM((NRING, BLK, DH), jnp.float32),
        pltpu.VMEM_SHARED((OUTP, DH), jnp.float32),
        pltpu.SemaphoreType.DMA((NRING,)),
        pltpu.SemaphoreType.DMA((NRING,)),
    ],
)
def _scatter_kernel(tab_hbm, ii_hbm, jj_hbm, vv_hbm, out_hbm,
                    ii_v, jj_v, vv_v, rows_v, acc_sh, gsem, ssem):
    c = lax.axis_index("c")
    s = lax.axis_index("s")
    coff = c * IN_N

    def _load_superblock(m):
        pltpu.sync_copy(ii_hbm.at[s * NSB + m], ii_v)
        pltpu.sync_copy(jj_hbm.at[s * NSB + m], jj_v)
        pltpu.sync_copy(vv_hbm.at[s * NSB + m], vv_v)

        def _adj(r, _):
            for g in range(BLK // 16):
                jj_v[r, pl.ds(g * 16, 16)] = jj_v[r, pl.ds(g * 16, 16)] + coff
            return 0
        lax.fori_loop(0, SBE, _adj, 0)

    def _drain_scatter(buf):
        pltpu.make_async_copy(
            rows_v.at[buf], acc_sh.at[ii_v.at[0]], ssem.at[buf]).wait()

    def _start_gather(lb, buf):
        pltpu.async_copy(tab_hbm.at[jj_v.at[lb]], rows_v.at[buf], gsem.at[buf])

    _load_superblock(0)

    zero16 = jnp.zeros((16,), jnp.float32)

    def _zrow(r, _):
        for g in range(DH // 16):
            rows_v[0, r, pl.ds(g * 16, 16)] = zero16
        return 0
    lax.fori_loop(0, BLK, _zrow, 0)
    for t in range(ROWS_T // BLK):
        pltpu.async_copy(rows_v.at[0],
                         acc_sh.at[pl.ds(s * ROWS_T + t * BLK, BLK)],
                         gsem.at[0])
    for t in range(ROWS_T // BLK):
        pltpu.make_async_copy(
            rows_v.at[0], acc_sh.at[pl.ds(s * ROWS_T, BLK)], gsem.at[0]).wait()
    plsc.subcore_barrier()

    _start_gather(0, 0)
    _start_gather(1, 1)

    def _body(b, _):
        lb = lax.rem(b, SBE)
        buf = lax.rem(b, NRING)
        nbuf = lax.rem(b + 1, NRING)

        @pl.when(jnp.logical_and(lb == 0, b > 0))
        def _boundary():
            for d in (2, 1):
                _drain_scatter(lax.rem(b - d, NRING))
            _load_superblock(lax.div(b, SBE))
            _start_gather(0, buf)
            _start_gather(1, nbuf)

        @pl.when(lb >= 2)
        def _():
            _drain_scatter(lax.rem(b + 2, NRING))

        @pl.when(jnp.logical_and(b + 2 < _NBLK2, lax.rem(b + 2, SBE) >= 2))
        def _():
            _start_gather(lax.rem(b + 2, SBE), lax.rem(b + 2, NRING))

        pltpu.make_async_copy(
            tab_hbm.at[jj_v.at[lb]], rows_v.at[buf], gsem.at[buf]).wait()

        for g in range(BLK // 16):
            vvec = vv_v[pl.ds(lb * BLK + g * 16, 16)]
            for l in range(16):
                vb = lax.gather(
                    vvec, jnp.full((16, 1), l, jnp.int32),
                    dimension_numbers=lax.GatherDimensionNumbers(
                        offset_dims=(), collapsed_slice_dims=(0,),
                        start_index_map=(0,)),
                    slice_sizes=(1,),
                    mode=lax.GatherScatterMode.PROMISE_IN_BOUNDS)
                e = g * 16 + l
                for k in range(DH // 16):
                    rows_v[buf, e, pl.ds(k * 16, 16)] = (
                        rows_v[buf, e, pl.ds(k * 16, 16)] * vb)

        pltpu.async_copy(rows_v.at[buf], acc_sh.at[ii_v.at[lb]], ssem.at[buf],
                         add=True)
        return 0
    lax.fori_loop(0, _NBLK2, _body, 0)

    _drain_scatter((_NBLK2 - 2) % NRING)
    _drain_scatter((_NBLK2 - 1) % NRING)
    plsc.subcore_barrier()

    def _row0(t):
        return pl.ds(s * ROWS_T + t * BLK, BLK)

    for t in range(NRING):
        pltpu.async_copy(acc_sh.at[_row0(t)], rows_v.at[t], gsem.at[t])
    for t in range(ROWS_T // BLK):
        sl = t % NRING
        pltpu.make_async_copy(
            acc_sh.at[_row0(t)], rows_v.at[sl], gsem.at[sl]).wait()
        pltpu.async_copy(rows_v.at[sl], out_hbm.at[c, _row0(t)], ssem.at[sl])
        if t + NRING < ROWS_T // BLK:
            pltpu.make_async_copy(
                rows_v.at[sl], out_hbm.at[c, _row0(t)], ssem.at[sl]).wait()
            pltpu.async_copy(acc_sh.at[_row0(t + NRING)], rows_v.at[sl],
                             gsem.at[sl])
    for t in range(ROWS_T // BLK - NRING, ROWS_T // BLK):
        sl = t % NRING
        pltpu.make_async_copy(
            rows_v.at[sl], out_hbm.at[c, _row0(t)], ssem.at[sl]).wait()


def kernel(input, params):
    u = jax.random.uniform(jax.random.key(12345), (KP, N_ADD, 2), dtype=jnp.float32)
    u_t = jnp.pad(u.transpose(1, 2, 0).reshape(2 * N_ADD, KP), ((0, 0), (0, PADK - KP)))
    p_t = jnp.pad(params.T, ((0, 0), (0, PADK - KP)))
    iN, jN, vN = _gen_entries(
        p_t.reshape(4, NBLK1, 128), u_t.reshape(2 * N_ADD, NBLK1, 128))
    iN, jN, vN = iN.reshape(-1), jN.reshape(-1), vN.reshape(-1)
    ii = jnp.concatenate([iN, jN]).reshape(N_TILES * NSB, SBE, BLK)
    jj = jnp.concatenate([jN, iN]).reshape(N_TILES * NSB, SBE, BLK)
    vv = jnp.concatenate([vN, vN]).reshape(N_TILES * NSB, SBE * BLK)
    tab = jnp.concatenate([input[:, :DH], input[:, DH:]], axis=0)
    out2 = _scatter_kernel(tab, ii, jj, vv)
    return jnp.concatenate([out2[0, :OUT_N], out2[1, :OUT_N]], axis=1)


# speedup vs baseline: 1.0085x; 1.0052x over previous
import functools

import jax
import jax.numpy as jnp
from jax import lax
from jax.experimental import pallas as pl
from jax.experimental.pallas import tpu as pltpu
from jax.experimental.pallas import tpu_sc as plsc

EPS = 1e-7
SIGMA_BOOST = 2.0
IN_N = 10000
OUT_N = 10000
KP = 25000
N_ADD = 4
SIGMA_SCALE = 0.2
D_IN = 256
DH = 128

PADK = 25600
NBLK1 = PADK // 128
NCAND = 8

NE = 2 * NCAND * PADK
N_TILES = 16
BLK = 64
OUTP = 10240
ROWS_T = OUTP // N_TILES


def _gen_body(p_ref, u_ref, i_ref, j_ref, v_ref):
    p0, p1, p2, p3 = p_ref[0], p_ref[1], p_ref[2], p_ref[3]
    m0 = jax.nn.sigmoid(p0) * (OUT_N - 1.0)
    m1 = jax.nn.sigmoid(p1) * (IN_N - 1.0)
    x = p2 + SIGMA_BOOST
    softplus = jnp.maximum(x, 0.0) + jnp.log1p(jnp.exp(-jnp.abs(x)))
    sig = (softplus + EPS) * (OUT_N * SIGMA_SCALE)
    inv = 1.0 / (EPS + sig)
    f0, c0 = jnp.floor(m0), jnp.ceil(m0)
    f1, c1 = jnp.floor(m1), jnp.ceil(m1)
    cands = [(f0, f1), (f0, c1), (c0, f1), (c0, c1)]
    for smp in range(N_ADD):
        u0 = u_ref[2 * smp] * (1.0 - EPS)
        u1 = u_ref[2 * smp + 1] * (1.0 - EPS)
        cands.append((jnp.floor(u0 * OUT_N), jnp.floor(u1 * IN_N)))
    props = []
    for d0, d1 in cands:
        dx, dy = d0 - m0, d1 - m1
        props.append(jnp.exp(-0.5 * (dx * dx + dy * dy) * inv))
    ssum = sum(props) + NCAND * EPS
    for d, ((d0, d1), pr) in enumerate(zip(cands, props)):
        val = p3 * pr / ssum
        val = jnp.where(d0 == d1, 0.0, val)
        i0 = d0.astype(jnp.int32)
        i1 = d1.astype(jnp.int32)
        i_ref[0, d] = i0
        i_ref[1, d] = i1
        j_ref[0, d] = i1
        j_ref[1, d] = i0
        v_ref[0, d] = val
        v_ref[1, d] = val


def _gen_entries(p_t, u_t):
    return pl.pallas_call(
        _gen_body,
        out_shape=(
            jax.ShapeDtypeStruct((2, NCAND, NBLK1, 128), jnp.int32),
            jax.ShapeDtypeStruct((2, NCAND, NBLK1, 128), jnp.int32),
            jax.ShapeDtypeStruct((2, NCAND, NBLK1, 128), jnp.float32),
        ),
    )(p_t, u_t)


_MESH = plsc.VectorSubcoreMesh(core_axis_name="c", subcore_axis_name="s")
_TILE_ENTRIES = NE // N_TILES
_NBLK2 = _TILE_ENTRIES // BLK
NRING = 4
SBE = 40
NSB = _NBLK2 // SBE


@functools.partial(
    pl.kernel,
    out_type=jax.ShapeDtypeStruct((2, OUTP, DH), jnp.float32),
    mesh=_MESH,
    scratch_types=[
        pltpu.VMEM((SBE, BLK), jnp.int32),
        pltpu.VMEM((SBE, BLK), jnp.int32),
        pltpu.VMEM((SBE * BLK,), jnp.float32),
        pltpu.VMEM((NRING, BLK, DH), jnp.float32),
        pltpu.VMEM_SHARED((OUTP, DH), jnp.float32),
        pltpu.SemaphoreType.DMA((NRING,)),
        pltpu.SemaphoreType.DMA((NRING,)),
    ],
)
def _scatter_kernel(tab_hbm, ii_hbm, jj_hbm, vv_hbm, out_hbm,
                    ii_v, jj_v, vv_v, rows_v, acc_sh, gsem, ssem):
    c = lax.axis_index("c")
    s = lax.axis_index("s")
    coff = c

    def _load_superblock(m):
        pltpu.sync_copy(ii_hbm.at[s * NSB + m], ii_v)
        pltpu.sync_copy(jj_hbm.at[s * NSB + m], jj_v)
        pltpu.sync_copy(vv_hbm.at[s * NSB + m], vv_v)

        def _adj(r, _):
            for g in range(BLK // 16):
                w = jj_v[r, pl.ds(g * 16, 16)]
                jj_v[r, pl.ds(g * 16, 16)] = w + w + coff
            return 0
        lax.fori_loop(0, SBE, _adj, 0)

    def _drain_scatter(buf):
        pltpu.make_async_copy(
            rows_v.at[buf], acc_sh.at[ii_v.at[0]], ssem.at[buf]).wait()

    def _start_gather(lb, buf):
        pltpu.async_copy(tab_hbm.at[jj_v.at[lb]], rows_v.at[buf], gsem.at[buf])

    _load_superblock(0)

    zero16 = jnp.zeros((16,), jnp.float32)

    def _zrow(r, _):
        for g in range(DH // 16):
            rows_v[0, r, pl.ds(g * 16, 16)] = zero16
        return 0
    lax.fori_loop(0, BLK, _zrow, 0)
    for t in range(ROWS_T // BLK):
        pltpu.async_copy(rows_v.at[0],
                         acc_sh.at[pl.ds(s * ROWS_T + t * BLK, BLK)],
                         gsem.at[0])
    for t in range(ROWS_T // BLK):
        pltpu.make_async_copy(
            rows_v.at[0], acc_sh.at[pl.ds(s * ROWS_T, BLK)], gsem.at[0]).wait()
    plsc.subcore_barrier()

    _start_gather(0, 0)
    _start_gather(1, 1)

    def _body(b, _):
        lb = lax.rem(b, SBE)
        buf = lax.rem(b, NRING)
        nbuf = lax.rem(b + 1, NRING)

        @pl.when(jnp.logical_and(lb == 0, b > 0))
        def _boundary():
            for d in (2, 1):
                _drain_scatter(lax.rem(b - d, NRING))
            _load_superblock(lax.div(b, SBE))
            _start_gather(0, buf)
            _start_gather(1, nbuf)

        @pl.when(lb >= 2)
        def _():
            _drain_scatter(lax.rem(b + 2, NRING))

        @pl.when(jnp.logical_and(b + 2 < _NBLK2, lax.rem(b + 2, SBE) >= 2))
        def _():
            _start_gather(lax.rem(b + 2, SBE), lax.rem(b + 2, NRING))

        pltpu.make_async_copy(
            tab_hbm.at[jj_v.at[lb]], rows_v.at[buf], gsem.at[buf]).wait()

        for g in range(BLK // 16):
            vvec = vv_v[pl.ds(lb * BLK + g * 16, 16)]
            for l in range(16):
                vb = lax.gather(
                    vvec, jnp.full((16, 1), l, jnp.int32),
                    dimension_numbers=lax.GatherDimensionNumbers(
                        offset_dims=(), collapsed_slice_dims=(0,),
                        start_index_map=(0,)),
                    slice_sizes=(1,),
                    mode=lax.GatherScatterMode.PROMISE_IN_BOUNDS)
                e = g * 16 + l
                for k in range(DH // 16):
                    rows_v[buf, e, pl.ds(k * 16, 16)] = (
                        rows_v[buf, e, pl.ds(k * 16, 16)] * vb)

        pltpu.async_copy(rows_v.at[buf], acc_sh.at[ii_v.at[lb]], ssem.at[buf],
                         add=True)
        return 0
    lax.fori_loop(0, _NBLK2, _body, 0)

    _drain_scatter((_NBLK2 - 2) % NRING)
    _drain_scatter((_NBLK2 - 1) % NRING)
    plsc.subcore_barrier()

    def _row0(t):
        return pl.ds(s * ROWS_T + t * BLK, BLK)

    for t in range(NRING):
        pltpu.async_copy(acc_sh.at[_row0(t)], rows_v.at[t], gsem.at[t])
    for t in range(ROWS_T // BLK):
        sl = t % NRING
        pltpu.make_async_copy(
            acc_sh.at[_row0(t)], rows_v.at[sl], gsem.at[sl]).wait()
        pltpu.async_copy(rows_v.at[sl], out_hbm.at[c, _row0(t)], ssem.at[sl])
        if t + NRING < ROWS_T // BLK:
            pltpu.make_async_copy(
                rows_v.at[sl], out_hbm.at[c, _row0(t)], ssem.at[sl]).wait()
            pltpu.async_copy(acc_sh.at[_row0(t + NRING)], rows_v.at[sl],
                             gsem.at[sl])
    for t in range(ROWS_T // BLK - NRING, ROWS_T // BLK):
        sl = t % NRING
        pltpu.make_async_copy(
            rows_v.at[sl], out_hbm.at[c, _row0(t)], ssem.at[sl]).wait()


def kernel(input, params):
    u = jax.random.uniform(jax.random.key(12345), (KP, N_ADD, 2), dtype=jnp.float32)
    u_t = jnp.pad(u.transpose(1, 2, 0).reshape(2 * N_ADD, KP), ((0, 0), (0, PADK - KP)))
    p_t = jnp.pad(params.T, ((0, 0), (0, PADK - KP)))
    iN, jN, vN = _gen_entries(
        p_t.reshape(4, NBLK1, 128), u_t.reshape(2 * N_ADD, NBLK1, 128))
    ii = iN.reshape(N_TILES * NSB, SBE, BLK)
    jj = jN.reshape(N_TILES * NSB, SBE, BLK)
    vv = vN.reshape(N_TILES * NSB, SBE * BLK)
    tab = input.reshape(2 * IN_N, DH)
    out2 = _scatter_kernel(tab, ii, jj, vv)
    return jnp.concatenate([out2[0, :OUT_N], out2[1, :OUT_N]], axis=1)
